# Initial kernel scaffold; baseline (speedup 1.0000x reference)
import functools

import jax
import jax.numpy as jnp
from jax import lax
from jax.experimental import pallas as pl
from jax.experimental.pallas import tpu as pltpu
from jax.experimental.pallas import tpu_sc as plsc

N_NODES = 10000
N_EDGES = 320000
D = 128

NC = 2
NS = 16
NW = NC * NS

CH = 128
NCHUNK = 80
EPT = NCHUNK * CH
E_PAD = NW * EPT
N_ACC = 10240
RPT = N_ACC // NS
ZR = 128

_MESH = plsc.VectorSubcoreMesh(core_axis_name="c", subcore_axis_name="s")


@functools.partial(
    pl.kernel,
    out_type=jax.ShapeDtypeStruct((NC, N_ACC), jnp.float32),
    mesh=_MESH,
    scratch_types=[
        pltpu.VMEM((NCHUNK, CH), jnp.int32),
        pltpu.VMEM((N_ACC,), jnp.float32),
        pltpu.VMEM((NS, RPT), jnp.float32),
        pltpu.VMEM_SHARED((NS, N_ACC), jnp.float32),
        pltpu.SemaphoreType.DMA,
    ],
)
def _deg_kernel(dst_hbm, out_hbm, dst_v, deg_v, red_v, stage_sh, sem):
    cid = lax.axis_index("c")
    sid = lax.axis_index("s")
    wid = cid * NS + sid

    pltpu.async_copy(dst_hbm.at[wid], dst_v, sem).wait()

    zero16 = jnp.zeros((16,), jnp.float32)
    one16 = jnp.ones((16,), jnp.float32)

    @pl.loop(0, N_ACC, step=16)
    def _(i):
        deg_v[pl.ds(i, 16)] = zero16

    @pl.loop(0, NCHUNK)
    def _(j):
        for k in range(CH // 16):
            idx = dst_v[j, pl.ds(k * 16, 16)]
            plsc.addupdate_scatter(deg_v, [idx], one16)

    pltpu.sync_copy(deg_v, stage_sh.at[sid])
    plsc.subcore_barrier()
    pltpu.sync_copy(stage_sh.at[:, pl.ds(sid * RPT, RPT)], red_v)

    @pl.loop(0, RPT, step=16)
    def _(c):
        acc = red_v[0, pl.ds(c, 16)]
        for r in range(1, NS):
            acc = acc + red_v[r, pl.ds(c, 16)]
        deg_v[pl.ds(c, 16)] = acc

    pltpu.sync_copy(deg_v.at[pl.ds(0, RPT)], out_hbm.at[cid, pl.ds(sid * RPT, RPT)])


@functools.partial(
    pl.kernel,
    out_type=jax.ShapeDtypeStruct((NC, N_ACC, D), jnp.float32),
    mesh=_MESH,
    scratch_types=[
        pltpu.VMEM((NCHUNK, CH), jnp.int32),
        pltpu.VMEM((NCHUNK, CH), jnp.int32),
        pltpu.VMEM((CH, D), jnp.float32),
        pltpu.VMEM((CH, D), jnp.float32),
        pltpu.VMEM((ZR, D), jnp.float32),
        pltpu.VMEM_SHARED((N_ACC, D), jnp.float32),
        pltpu.SemaphoreType.DMA,
        pltpu.SemaphoreType.DMA,
        pltpu.SemaphoreType.DMA,
    ],
)
def _edge_kernel(g_hbm, src_hbm, dst_hbm, out_hbm,
                 src_v, dst_v, rows_a, rows_b, zero_v, acc_sh,
                 sem_i, sem_a, sem_b):
    cid = lax.axis_index("c")
    sid = lax.axis_index("s")
    wid = cid * NS + sid

    pltpu.async_copy(src_hbm.at[wid], src_v, sem_i).wait()
    pltpu.async_copy(dst_hbm.at[wid], dst_v, sem_i).wait()

    zero16 = jnp.zeros((16,), jnp.float32)

    @pl.loop(0, ZR)
    def _(i):
        for k in range(D // 16):
            zero_v[i, pl.ds(k * 16, 16)] = zero16

    @pl.loop(0, RPT, step=ZR)
    def _(r):
        pltpu.sync_copy(zero_v, acc_sh.at[pl.ds(sid * RPT + r, ZR)])

    plsc.subcore_barrier()

    pltpu.async_copy(g_hbm.at[src_v.at[0]], rows_a, sem_a)

    @pl.loop(0, NCHUNK, step=2)
    def _(j):
        pltpu.async_copy(g_hbm.at[src_v.at[j + 1]], rows_b, sem_b)
        pltpu.make_async_copy(g_hbm.at[src_v.at[j]], rows_a, sem_a).wait()
        pltpu.sync_copy(rows_a, acc_sh.at[dst_v.at[j]], add=True)

        @pl.when(j + 2 < NCHUNK)
        def _():
            pltpu.async_copy(g_hbm.at[src_v.at[j + 2]], rows_a, sem_a)

        pltpu.make_async_copy(g_hbm.at[src_v.at[j + 1]], rows_b, sem_b).wait()
        pltpu.sync_copy(rows_b, acc_sh.at[dst_v.at[j + 1]], add=True)

    plsc.subcore_barrier()
    pltpu.sync_copy(acc_sh.at[pl.ds(sid * RPT, RPT)],
                    out_hbm.at[cid, pl.ds(sid * RPT, RPT)])


_RB = 1000
_GRID = N_NODES // _RB


def _tc1_body(x_ref, w_ref, d0_ref, d1_ref, g_ref, dinv_ref):
    dinv = lax.rsqrt(d0_ref[...] + d1_ref[...] + 1.0)
    g_ref[...] = jnp.dot(x_ref[...], w_ref[...],
                         preferred_element_type=jnp.float32) * dinv
    dinv_ref[...] = dinv


def _tc2_body(acc_ref, g1_ref, dinv_ref, w_ref, b_ref, g2_ref):
    dinv = dinv_ref[...]
    h = (acc_ref[0] + acc_ref[1]) * dinv + g1_ref[...] * dinv + b_ref[...]
    h = jnp.where(h > 0, h, jnp.expm1(h))
    g2_ref[...] = jnp.dot(h, w_ref[...],
                          preferred_element_type=jnp.float32) * dinv


def _tc3_body(acc_ref, g2_ref, dinv_ref, b_ref, o_ref):
    dinv = dinv_ref[...]
    h = (acc_ref[0] + acc_ref[1]) * dinv + g2_ref[...] * dinv + b_ref[...]
    o_ref[...] = jax.nn.sigmoid(h)


_row_spec = pl.BlockSpec((_RB, D), lambda i: (i, 0))
_col_spec = pl.BlockSpec((_RB, 1), lambda i: (i, 0))
_w_spec = pl.BlockSpec((D, D), lambda i: (0, 0))
_b_spec = pl.BlockSpec((1, D), lambda i: (0, 0))
_acc_spec = pl.BlockSpec((NC, _RB, D), lambda i: (0, i, 0))

_tc1 = pl.pallas_call(
    _tc1_body,
    grid=(_GRID,),
    in_specs=[_row_spec, _w_spec, _col_spec, _col_spec],
    out_specs=[_row_spec, _col_spec],
    out_shape=[jax.ShapeDtypeStruct((N_NODES, D), jnp.float32),
               jax.ShapeDtypeStruct((N_NODES, 1), jnp.float32)],
)

_tc2 = pl.pallas_call(
    _tc2_body,
    grid=(_GRID,),
    in_specs=[_acc_spec, _row_spec, _col_spec, _w_spec, _b_spec],
    out_specs=_row_spec,
    out_shape=jax.ShapeDtypeStruct((N_NODES, D), jnp.float32),
)

_tc3 = pl.pallas_call(
    _tc3_body,
    grid=(_GRID,),
    in_specs=[_acc_spec, _row_spec, _col_spec, _b_spec],
    out_specs=_row_spec,
    out_shape=jax.ShapeDtypeStruct((N_NODES, D), jnp.float32),
)


@jax.jit
def kernel(x, edge_index, edge_attr, W1, b1, W2, b2):
    del edge_attr
    src = edge_index[0].astype(jnp.int32)
    dst = edge_index[1].astype(jnp.int32)
    pad = E_PAD - N_EDGES
    src3 = jnp.pad(src, (0, pad)).reshape(NW, NCHUNK, CH)
    dst3 = jnp.pad(dst, (0, pad), constant_values=N_NODES).reshape(NW, NCHUNK, CH)

    deg = _deg_kernel(dst3)
    d0 = deg[0, :N_NODES, None]
    d1 = deg[1, :N_NODES, None]

    g1, dinv = _tc1(x, W1, d0, d1)
    acc1 = _edge_kernel(g1, src3, dst3)
    g2 = _tc2(acc1, g1, dinv, W2, b1.reshape(1, D))
    acc2 = _edge_kernel(g2, src3, dst3)
    return _tc3(acc2, g2, dinv, b2.reshape(1, D))


# speedup vs baseline: 7.6773x; 7.6773x over previous
import dataclasses
import functools

import jax
import jax.numpy as jnp
from jax import lax
from jax.experimental import pallas as pl
from jax.experimental.pallas import tpu as pltpu
from jax.experimental.pallas import tpu_sc as plsc

N_NODES = 10000
N_EDGES = 320000
D = 128

NC = 2
NS = 16
NW = NC * NS

CH = 128
NCHUNK = 80
EPT = NCHUNK * CH
E_PAD = NW * EPT
N_ACC = 10240
RPT = N_ACC // NS

FPT = D // NW
CHE = 4096
NCHE = E_PAD // CHE
UNROLL = 8

_MESH = plsc.VectorSubcoreMesh(core_axis_name="c", subcore_axis_name="s",
                               num_cores=NC, num_subcores=NS)

_SC_PARAMS = pltpu.CompilerParams()
if "needs_layout_passes" in pltpu.CompilerParams.__dataclass_fields__:
    _SC_PARAMS = dataclasses.replace(_SC_PARAMS, needs_layout_passes=False)


@functools.partial(
    pl.kernel,
    out_type=jax.ShapeDtypeStruct((NC, N_ACC), jnp.float32),
    mesh=_MESH,
    scratch_types=[
        pltpu.VMEM((NCHUNK, CH), jnp.int32),
        pltpu.VMEM((N_ACC,), jnp.float32),
        pltpu.VMEM((NS, RPT), jnp.float32),
        pltpu.VMEM_SHARED((NS, N_ACC), jnp.float32),
        pltpu.SemaphoreType.DMA,
    ],
    compiler_params=_SC_PARAMS,
)
def _deg_kernel(dst_hbm, out_hbm, dst_v, deg_v, red_v, stage_sh, sem):
    cid = lax.axis_index("c")
    sid = lax.axis_index("s")
    wid = cid * NS + sid

    pltpu.async_copy(dst_hbm.at[wid], dst_v, sem).wait()

    zero16 = jnp.zeros((16,), jnp.float32)
    one16 = jnp.ones((16,), jnp.float32)

    @pl.loop(0, N_ACC, step=16)
    def _(i):
        deg_v[pl.ds(i, 16)] = zero16

    @pl.loop(0, NCHUNK)
    def _(j):
        for k in range(CH // 16):
            idx = dst_v[j, pl.ds(k * 16, 16)]
            plsc.addupdate_scatter(deg_v, [idx], one16)

    pltpu.sync_copy(deg_v, stage_sh.at[sid])
    plsc.subcore_barrier()
    for r in range(NS):
        pltpu.sync_copy(stage_sh.at[r, pl.ds(sid * RPT, RPT)], red_v.at[r])

    @pl.loop(0, RPT, step=16)
    def _(c):
        acc = red_v[0, pl.ds(c, 16)]
        for r in range(1, NS):
            acc = acc + red_v[r, pl.ds(c, 16)]
        deg_v[pl.ds(c, 16)] = acc

    pltpu.sync_copy(deg_v.at[pl.ds(0, RPT)], out_hbm.at[cid, pl.ds(sid * RPT, RPT)])


@functools.partial(
    pl.kernel,
    out_type=jax.ShapeDtypeStruct((D * N_ACC,), jnp.float32),
    mesh=_MESH,
    scratch_types=[
        pltpu.VMEM((FPT * N_ACC,), jnp.float32),
        pltpu.VMEM((FPT * N_ACC,), jnp.float32),
        pltpu.VMEM((2, CHE), jnp.int32),
        pltpu.VMEM((2, CHE), jnp.int32),
        pltpu.SemaphoreType.DMA,
        pltpu.SemaphoreType.DMA,
        pltpu.SemaphoreType.DMA,
    ],
    compiler_params=_SC_PARAMS,
)
def _edge_kernel(gt_hbm, idx_hbm, out_hbm, slab_v, acc_v, idx_a, idx_b,
                 sem_s, sem_a, sem_b):
    cid = lax.axis_index("c")
    sid = lax.axis_index("s")
    wid = cid * NS + sid

    pltpu.async_copy(gt_hbm.at[pl.ds(wid * FPT * N_ACC, FPT * N_ACC)],
                     slab_v, sem_s).wait()

    zero16 = jnp.zeros((16,), jnp.float32)

    @pl.loop(0, FPT * N_ACC, step=16)
    def _(i):
        acc_v[pl.ds(i, 16)] = zero16

    off16 = jnp.full((16,), N_ACC, jnp.int32)

    def process(idx_v):
        @pl.loop(0, CHE, step=16 * UNROLL)
        def _(e):
            for u in range(UNROLL):
                srcv = idx_v[0, pl.ds(e + u * 16, 16)]
                dstv = idx_v[1, pl.ds(e + u * 16, 16)]
                for r in range(FPT):
                    vals = plsc.load_gather(slab_v, [srcv])
                    plsc.addupdate_scatter(acc_v, [dstv], vals)
                    if r + 1 < FPT:
                        srcv = srcv + off16
                        dstv = dstv + off16

    @pl.loop(0, NCHE, step=2)
    def _(j):
        da = pltpu.async_copy(idx_hbm.at[j], idx_a, sem_a)
        db = pltpu.async_copy(idx_hbm.at[j + 1], idx_b, sem_b)
        da.wait()
        process(idx_a)
        db.wait()
        process(idx_b)

    pltpu.sync_copy(acc_v, out_hbm.at[pl.ds(wid * FPT * N_ACC, FPT * N_ACC)])


_CB = 1024
_GRID = N_ACC // _CB


def _tc1_body(x_ref, w_ref, d0_ref, d1_ref, g_ref, dinv_ref):
    dinv = lax.rsqrt(d0_ref[...] + d1_ref[...] + 1.0)
    gt = lax.dot_general(w_ref[...], x_ref[...],
                         (((0,), (1,)), ((), ())),
                         preferred_element_type=jnp.float32)
    g_ref[...] = gt * dinv
    dinv_ref[...] = dinv


def _tc2_body(acc_ref, g1_ref, dinv_ref, w_ref, b_ref, g2_ref):
    dinv = dinv_ref[...]
    h = (acc_ref[...] + g1_ref[...]) * dinv + b_ref[...]
    h = jnp.where(h > 0, h, jnp.exp(jnp.minimum(h, 0.0)) - 1.0)
    gt = lax.dot_general(w_ref[...], h,
                         (((0,), (0,)), ((), ())),
                         preferred_element_type=jnp.float32)
    g2_ref[...] = gt * dinv


def _tc3_body(acc_ref, g2_ref, dinv_ref, b_ref, o_ref):
    h = (acc_ref[...] + g2_ref[...]) * dinv_ref[...] + b_ref[...]
    o_ref[...] = jax.nn.sigmoid(h)


_gt_spec = pl.BlockSpec((D, _CB), lambda i: (0, i))
_x_spec = pl.BlockSpec((_CB, D), lambda i: (i, 0))
_row_spec = pl.BlockSpec((1, _CB), lambda i: (0, i))
_w_spec = pl.BlockSpec((D, D), lambda i: (0, 0))
_b_spec = pl.BlockSpec((D, 1), lambda i: (0, 0))
_acc_spec = pl.BlockSpec((D, _CB), lambda i: (0, i))

_tc1 = pl.pallas_call(
    _tc1_body,
    grid=(_GRID,),
    in_specs=[_x_spec, _w_spec, _row_spec, _row_spec],
    out_specs=[_gt_spec, _row_spec],
    out_shape=[jax.ShapeDtypeStruct((D, N_ACC), jnp.float32),
               jax.ShapeDtypeStruct((1, N_ACC), jnp.float32)],
)

_tc2 = pl.pallas_call(
    _tc2_body,
    grid=(_GRID,),
    in_specs=[_acc_spec, _gt_spec, _row_spec, _w_spec, _b_spec],
    out_specs=_gt_spec,
    out_shape=jax.ShapeDtypeStruct((D, N_ACC), jnp.float32),
)

_tc3 = pl.pallas_call(
    _tc3_body,
    grid=(_GRID,),
    in_specs=[_acc_spec, _gt_spec, _row_spec, _b_spec],
    out_specs=_gt_spec,
    out_shape=jax.ShapeDtypeStruct((D, N_ACC), jnp.float32),
)


@jax.jit
def kernel(x, edge_index, edge_attr, W1, b1, W2, b2):
    del edge_attr
    src = edge_index[0].astype(jnp.int32)
    dst = edge_index[1].astype(jnp.int32)
    pad = E_PAD - N_EDGES
    src_p = jnp.pad(src, (0, pad))
    dst_p = jnp.pad(dst, (0, pad), constant_values=N_NODES)
    dst3 = dst_p.reshape(NW, NCHUNK, CH)
    idx_pairs = jnp.stack([src_p.reshape(NCHE, CHE),
                           dst_p.reshape(NCHE, CHE)], axis=1)

    deg = _deg_kernel(dst3)
    d0 = deg[0].reshape(1, N_ACC)
    d1 = deg[1].reshape(1, N_ACC)

    x_p = jnp.pad(x, ((0, N_ACC - N_NODES), (0, 0)))
    g1, dinv = _tc1(x_p, W1, d0, d1)
    acc1 = _edge_kernel(g1.reshape(-1), idx_pairs).reshape(D, N_ACC)
    g2 = _tc2(acc1, g1, dinv, W2, b1.reshape(D, 1))
    acc2 = _edge_kernel(g2.reshape(-1), idx_pairs).reshape(D, N_ACC)
    out_t = _tc3(acc2, g2, dinv, b2.reshape(D, 1))
    return out_t[:, :N_NODES].T


# speedup vs baseline: 10.6643x; 1.3891x over previous
import dataclasses
import functools

import jax
import jax.numpy as jnp
from jax import lax
from jax.experimental import pallas as pl
from jax.experimental.pallas import tpu as pltpu
from jax.experimental.pallas import tpu_sc as plsc

N_NODES = 10000
N_EDGES = 320000
D = 128

NC = 2
NS = 16
NW = NC * NS

CH = 128
NCHUNK = 80
EPT = NCHUNK * CH
E_PAD = NW * EPT
N_ACC = 10240
RPT = N_ACC // NS

FPT = D // NW
CHE = 4096
NCHE = E_PAD // CHE
UNROLL = 8

_MESH = plsc.VectorSubcoreMesh(core_axis_name="c", subcore_axis_name="s",
                               num_cores=NC, num_subcores=NS)

_SC_PARAMS = pltpu.CompilerParams()
if "needs_layout_passes" in pltpu.CompilerParams.__dataclass_fields__:
    _SC_PARAMS = dataclasses.replace(_SC_PARAMS, needs_layout_passes=False)


@functools.partial(
    pl.kernel,
    out_type=jax.ShapeDtypeStruct((NC, N_ACC), jnp.float32),
    mesh=_MESH,
    scratch_types=[
        pltpu.VMEM((NCHUNK, CH), jnp.int32),
        pltpu.VMEM((N_ACC,), jnp.float32),
        pltpu.VMEM((NS, RPT), jnp.float32),
        pltpu.VMEM_SHARED((NS, N_ACC), jnp.float32),
        pltpu.SemaphoreType.DMA,
    ],
    compiler_params=_SC_PARAMS,
)
def _deg_kernel(dst_hbm, out_hbm, dst_v, deg_v, red_v, stage_sh, sem):
    cid = lax.axis_index("c")
    sid = lax.axis_index("s")
    wid = cid * NS + sid

    pltpu.async_copy(dst_hbm.at[wid], dst_v, sem).wait()

    zero16 = jnp.zeros((16,), jnp.float32)
    one16 = jnp.ones((16,), jnp.float32)

    @pl.loop(0, N_ACC, step=16)
    def _(i):
        deg_v[pl.ds(i, 16)] = zero16

    @pl.loop(0, NCHUNK)
    def _(j):
        for k in range(CH // 16):
            idx = dst_v[j, pl.ds(k * 16, 16)]
            plsc.addupdate_scatter(deg_v, [idx], one16)

    pltpu.sync_copy(deg_v, stage_sh.at[sid])
    plsc.subcore_barrier()
    for r in range(NS):
        pltpu.sync_copy(stage_sh.at[r, pl.ds(sid * RPT, RPT)], red_v.at[r])

    @pl.loop(0, RPT, step=16)
    def _(c):
        acc = red_v[0, pl.ds(c, 16)]
        for r in range(1, NS):
            acc = acc + red_v[r, pl.ds(c, 16)]
        deg_v[pl.ds(c, 16)] = acc

    pltpu.sync_copy(deg_v.at[pl.ds(0, RPT)], out_hbm.at[cid, pl.ds(sid * RPT, RPT)])


@functools.partial(
    pl.kernel,
    out_type=jax.ShapeDtypeStruct((D * N_ACC,), jnp.float32),
    mesh=_MESH,
    scratch_types=[
        pltpu.VMEM((FPT * N_ACC,), jnp.float32),
        pltpu.VMEM((FPT * N_ACC,), jnp.float32),
        pltpu.VMEM((2, CHE), jnp.int32),
        pltpu.VMEM((2, CHE), jnp.int32),
        pltpu.SemaphoreType.DMA,
        pltpu.SemaphoreType.DMA,
        pltpu.SemaphoreType.DMA,
    ],
    compiler_params=_SC_PARAMS,
)
def _edge_kernel(gt_hbm, idx_hbm, out_hbm, slab_v, acc_v, idx_a, idx_b,
                 sem_s, sem_a, sem_b):
    cid = lax.axis_index("c")
    sid = lax.axis_index("s")
    wid = cid * NS + sid

    pltpu.async_copy(gt_hbm.at[pl.ds(wid * FPT * N_ACC, FPT * N_ACC)],
                     slab_v, sem_s).wait()

    zero16 = jnp.zeros((16,), jnp.float32)

    @pl.loop(0, FPT * N_ACC, step=16)
    def _(i):
        acc_v[pl.ds(i, 16)] = zero16

    off16 = jnp.full((16,), N_ACC, jnp.int32)

    def process(idx_v):
        @pl.loop(0, CHE, step=16 * UNROLL)
        def _(e):
            for u in range(UNROLL):
                srcv = idx_v[0, pl.ds(e + u * 16, 16)]
                dstv = idx_v[1, pl.ds(e + u * 16, 16)]
                srcs, dsts = [srcv], [dstv]
                for r in range(1, FPT):
                    srcs.append(srcs[-1] + off16)
                    dsts.append(dsts[-1] + off16)
                vals = [plsc.load_gather(slab_v, [s]) for s in srcs]
                for r in range(FPT):
                    plsc.addupdate_scatter(acc_v, [dsts[r]], vals[r])

    @pl.loop(0, NCHE, step=2)
    def _(j):
        da = pltpu.async_copy(idx_hbm.at[j], idx_a, sem_a)
        db = pltpu.async_copy(idx_hbm.at[j + 1], idx_b, sem_b)
        da.wait()
        process(idx_a)
        db.wait()
        process(idx_b)

    pltpu.sync_copy(acc_v, out_hbm.at[pl.ds(wid * FPT * N_ACC, FPT * N_ACC)])


_CB = 1024
_GRID = N_ACC // _CB


def _tc1_body(x_ref, w_ref, d0_ref, d1_ref, g_ref, dinv_ref):
    dinv = lax.rsqrt(d0_ref[...] + d1_ref[...] + 1.0)
    gt = lax.dot_general(w_ref[...], x_ref[...],
                         (((0,), (1,)), ((), ())),
                         preferred_element_type=jnp.float32)
    g_ref[...] = gt * dinv
    dinv_ref[...] = dinv


def _tc2_body(acc_ref, g1_ref, dinv_ref, w_ref, b_ref, g2_ref):
    dinv = dinv_ref[...]
    h = (acc_ref[...] + g1_ref[...]) * dinv + b_ref[...]
    h = jnp.where(h > 0, h, jnp.exp(jnp.minimum(h, 0.0)) - 1.0)
    gt = lax.dot_general(w_ref[...], h,
                         (((0,), (0,)), ((), ())),
                         preferred_element_type=jnp.float32)
    g2_ref[...] = gt * dinv


def _tc3_body(acc_ref, g2_ref, dinv_ref, b_ref, o_ref):
    h = (acc_ref[...] + g2_ref[...]) * dinv_ref[...] + b_ref[...]
    o_ref[...] = jax.nn.sigmoid(h)


_gt_spec = pl.BlockSpec((D, _CB), lambda i: (0, i))
_x_spec = pl.BlockSpec((_CB, D), lambda i: (i, 0))
_row_spec = pl.BlockSpec((1, _CB), lambda i: (0, i))
_w_spec = pl.BlockSpec((D, D), lambda i: (0, 0))
_b_spec = pl.BlockSpec((D, 1), lambda i: (0, 0))
_acc_spec = pl.BlockSpec((D, _CB), lambda i: (0, i))

_tc1 = pl.pallas_call(
    _tc1_body,
    grid=(_GRID,),
    in_specs=[_x_spec, _w_spec, _row_spec, _row_spec],
    out_specs=[_gt_spec, _row_spec],
    out_shape=[jax.ShapeDtypeStruct((D, N_ACC), jnp.float32),
               jax.ShapeDtypeStruct((1, N_ACC), jnp.float32)],
)

_tc2 = pl.pallas_call(
    _tc2_body,
    grid=(_GRID,),
    in_specs=[_acc_spec, _gt_spec, _row_spec, _w_spec, _b_spec],
    out_specs=_gt_spec,
    out_shape=jax.ShapeDtypeStruct((D, N_ACC), jnp.float32),
)

_tc3 = pl.pallas_call(
    _tc3_body,
    grid=(_GRID,),
    in_specs=[_acc_spec, _gt_spec, _row_spec, _b_spec],
    out_specs=_gt_spec,
    out_shape=jax.ShapeDtypeStruct((D, N_ACC), jnp.float32),
)


@jax.jit
def kernel(x, edge_index, edge_attr, W1, b1, W2, b2):
    del edge_attr
    src = edge_index[0].astype(jnp.int32)
    dst = edge_index[1].astype(jnp.int32)
    pad = E_PAD - N_EDGES
    src_p = jnp.pad(src, (0, pad))
    dst_p = jnp.pad(dst, (0, pad), constant_values=N_NODES)
    dst3 = dst_p.reshape(NW, NCHUNK, CH)
    idx_pairs = jnp.stack([src_p.reshape(NCHE, CHE),
                           dst_p.reshape(NCHE, CHE)], axis=1)

    deg = _deg_kernel(dst3)
    d0 = deg[0].reshape(1, N_ACC)
    d1 = deg[1].reshape(1, N_ACC)

    x_p = jnp.pad(x, ((0, N_ACC - N_NODES), (0, 0)))
    g1, dinv = _tc1(x_p, W1, d0, d1)
    acc1 = _edge_kernel(g1.reshape(-1), idx_pairs).reshape(D, N_ACC)
    g2 = _tc2(acc1, g1, dinv, W2, b1.reshape(D, 1))
    acc2 = _edge_kernel(g2.reshape(-1), idx_pairs).reshape(D, N_ACC)
    out_t = _tc3(acc2, g2, dinv, b2.reshape(D, 1))
    return out_t[:, :N_NODES].T
